# Initial kernel scaffold; baseline (speedup 1.0000x reference)
import jax
import jax.numpy as jnp
from jax.experimental import pallas as pl


def kernel(embed, label):
    raise NotImplementedError("write your pallas kernel here")


# speedup vs baseline: 1.2581x; 1.2581x over previous
import functools

import jax
import jax.numpy as jnp
from jax import lax
from jax.experimental import pallas as pl
from jax.experimental.pallas import tpu as pltpu
from jax.experimental.pallas import tpu_sc as plsc

W_PULL = 1.0
W_PUSH = 1.0
W_NORM = 0.001
W_EDGE = 10.0
D_V = 0.5
D_D = 1.5

B = 2
E = 16
Z = 64
PLANE = 64 * 64
NC = 2
NS = 16
NW = NC * NS
ZPW = Z // NS
NCHUNK = PLANE // 16
LPAD = 64
SUMS = 9 * 16
MISC = 2 * 16


def _nsqrt(x):
    i = plsc.bitcast(x, jnp.int32)
    i = jnp.int32(0x5F3759DF) - lax.shift_right_logical(i, 1)
    z = plsc.bitcast(i, jnp.float32)
    z = z * (1.5 - 0.5 * x * z * z)
    z = z * (1.5 - 0.5 * x * z * z)
    z = z * (1.5 - 0.5 * x * z * z)
    return x * z


def _mesh():
    return plsc.VectorSubcoreMesh(
        core_axis_name="c", subcore_axis_name="s", num_cores=NC, num_subcores=NS)


@functools.partial(
    pl.kernel,
    out_type=[
        jax.ShapeDtypeStruct((NW * SUMS,), jnp.float32),
        jax.ShapeDtypeStruct((NW * MISC,), jnp.float32),
        jax.ShapeDtypeStruct((B * Z * PLANE,), jnp.float32),
    ],
    mesh=_mesh(),
    compiler_params=pltpu.CompilerParams(needs_layout_passes=False),
    scratch_types=[
        pltpu.VMEM((PLANE + 2 * LPAD,), jnp.int32),
        pltpu.VMEM((PLANE + 2 * LPAD,), jnp.int32),
        pltpu.VMEM((PLANE + 2 * LPAD,), jnp.int32),
        pltpu.VMEM((PLANE,), jnp.int32),
        pltpu.VMEM((PLANE,), jnp.int32),
        pltpu.VMEM((PLANE,), jnp.int32),
        pltpu.VMEM((PLANE,), jnp.int32),
        pltpu.VMEM((PLANE,), jnp.int32),
        pltpu.VMEM((PLANE,), jnp.int32),
        pltpu.VMEM((PLANE,), jnp.int32),
        pltpu.VMEM((PLANE,), jnp.int32),
        pltpu.VMEM((PLANE,), jnp.float32),
        pltpu.VMEM((E * PLANE,), jnp.float32),
        pltpu.VMEM((SUMS,), jnp.float32),
        pltpu.VMEM((MISC,), jnp.float32),
        pltpu.SemaphoreType.DMA,
    ],
)
def _kern_a(embed, label, sums_out, misc_out, w_out,
            lab0, lab1, lab2, mn0, mn1, mn2, mx0, mx1, mx2, mnx, mxx,
            wbuf, emb, sums_acc, misc_acc, sem):
    labs = [lab0, lab1, lab2]
    mns = [mn0, mn1, mn2]
    mxs = [mx0, mx1, mx2]
    c = lax.axis_index("c")
    s = lax.axis_index("s")
    wid = c * NS + s
    b = c
    z0 = s * ZPW
    io = lax.iota(jnp.int32, 16)
    zf = jnp.zeros((16,), jnp.float32)
    onef = jnp.ones((16,), jnp.float32)

    for r in range(SUMS // 16):
        sums_acc[pl.ds(r * 16, 16)] = zf
    misc_acc[pl.ds(0, 16)] = zf
    misc_acc[pl.ds(16, 16)] = zf

    def load_lab(z, slot):
        zc = jnp.clip(z, 0, Z - 1)
        pltpu.sync_copy(label.at[pl.ds((b * Z + zc) * PLANE, PLANE)],
                        labs[slot].at[pl.ds(LPAD, PLANE)])

    def xy_pass(slot):
        lab = labs[slot]
        mnr = mns[slot]
        mxr = mxs[slot]

        def xrow(r, _):
            base = LPAD + r * 64
            for p in range(4):
                o = base + p * 16
                cv = lab[pl.ds(o, 16)]
                lv = lab[pl.ds(o - 1, 16)]
                rv = lab[pl.ds(o + 1, 16)]
                if p == 0:
                    lv = jnp.where(io == 0, cv, lv)
                if p == 3:
                    rv = jnp.where(io == 15, cv, rv)
                oo = r * 64 + p * 16
                mnx[pl.ds(oo, 16)] = jnp.minimum(jnp.minimum(lv, cv), rv)
                mxx[pl.ds(oo, 16)] = jnp.maximum(jnp.maximum(lv, cv), rv)
            return 0
        lax.fori_loop(0, 64, xrow, 0)

        def yrow(r, _):
            rm = jnp.maximum(r - 1, 0) * 64
            rc = r * 64
            rp = jnp.minimum(r + 1, 63) * 64
            for p in range(4):
                q = p * 16
                mnr[pl.ds(rc + q, 16)] = jnp.minimum(
                    jnp.minimum(mnx[pl.ds(rm + q, 16)], mnx[pl.ds(rc + q, 16)]),
                    mnx[pl.ds(rp + q, 16)])
                mxr[pl.ds(rc + q, 16)] = jnp.maximum(
                    jnp.maximum(mxx[pl.ds(rm + q, 16)], mxx[pl.ds(rc + q, 16)]),
                    mxx[pl.ds(rp + q, 16)])
            return 0
        lax.fori_loop(0, 64, yrow, 0)

    load_lab(z0 - 1, 0)
    xy_pass(0)
    load_lab(z0, 1)
    xy_pass(1)

    for k in range(ZPW):
        z = z0 + k
        sl_prev = k % 3
        sl_cur = (k + 1) % 3
        sl_next = (k + 2) % 3

        cps = [pltpu.async_copy(
                   embed.at[pl.ds(((b * E + e) * Z + z) * PLANE, PLANE)],
                   emb.at[pl.ds(e * PLANE, PLANE)], sem)
               for e in range(E)]

        load_lab(z + 1, sl_next)
        xy_pass(sl_next)

        mnp_, mnc_, mnn_ = mns[sl_prev], mns[sl_cur], mns[sl_next]
        mxp_, mxc_, mxn_ = mxs[sl_prev], mxs[sl_cur], mxs[sl_next]

        def wchunk(i, _):
            o = i * 16
            mnv = jnp.minimum(
                jnp.minimum(mnp_[pl.ds(o, 16)], mnc_[pl.ds(o, 16)]),
                mnn_[pl.ds(o, 16)])
            mxv = jnp.maximum(
                jnp.maximum(mxp_[pl.ds(o, 16)], mxc_[pl.ds(o, 16)]),
                mxn_[pl.ds(o, 16)])
            wbuf[pl.ds(o, 16)] = jnp.where(mxv != mnv, W_EDGE, 1.0)
            return 0
        lax.fori_loop(0, NCHUNK, wchunk, 0)

        pltpu.sync_copy(wbuf, w_out.at[pl.ds((b * Z + z) * PLANE, PLANE)])
        for cp in cps:
            cp.wait()

        labc_ = labs[sl_cur]

        def achunk(i, _):
            o = i * 16
            labv = labc_[pl.ds(LPAD + o, 16)]
            wv = wbuf[pl.ds(o, 16)]
            plsc.addupdate_scatter(misc_acc, [labv], wv)
            plsc.addupdate_scatter(misc_acc, [labv + 16], onef)
            lb16 = labv * 16
            for e in range(E):
                v = emb[pl.ds(e * PLANE + o, 16)]
                plsc.addupdate_scatter(sums_acc, [lb16 + e], wv * v)
            return 0
        lax.fori_loop(0, NCHUNK, achunk, 0)

    pltpu.sync_copy(sums_acc, sums_out.at[pl.ds(wid * SUMS, SUMS)])
    pltpu.sync_copy(misc_acc, misc_out.at[pl.ds(wid * MISC, MISC)])


@functools.partial(
    pl.kernel,
    out_type=[
        jax.ShapeDtypeStruct((NW * 16,), jnp.float32),
        jax.ShapeDtypeStruct((B * SUMS,), jnp.float32),
        jax.ShapeDtypeStruct((B * MISC,), jnp.float32),
    ],
    mesh=_mesh(),
    compiler_params=pltpu.CompilerParams(needs_layout_passes=False),
    scratch_types=[
        pltpu.VMEM((NS * SUMS,), jnp.float32),
        pltpu.VMEM((NS * MISC,), jnp.float32),
        pltpu.VMEM((SUMS,), jnp.float32),
        pltpu.VMEM((MISC,), jnp.float32),
        pltpu.VMEM((PLANE,), jnp.int32),
        pltpu.VMEM((PLANE,), jnp.float32),
        pltpu.VMEM((E * PLANE,), jnp.float32),
        pltpu.VMEM((16,), jnp.float32),
        pltpu.SemaphoreType.DMA,
    ],
)
def _kern_b(embed, label, w_scr, sums_p, misc_p,
            pull_out, centers_out, miscred_out,
            part_buf, mpart_buf, centers_v, miscred_v, lab_buf, wbuf, emb,
            pull_acc, sem):
    c = lax.axis_index("c")
    s = lax.axis_index("s")
    wid = c * NS + s
    b = c
    z0 = s * ZPW
    zf = jnp.zeros((16,), jnp.float32)

    pltpu.sync_copy(sums_p.at[pl.ds(b * NS * SUMS, NS * SUMS)], part_buf)
    pltpu.sync_copy(misc_p.at[pl.ds(b * NS * MISC, NS * MISC)], mpart_buf)

    for uid in range(9):
        def red(i, acc, _uid=uid):
            return acc + part_buf[pl.ds(i * SUMS + _uid * 16, 16)]
        centers_v[pl.ds(uid * 16, 16)] = lax.fori_loop(0, NS, red, zf)
    for r in range(2):
        def redm(i, acc, _r=r):
            return acc + mpart_buf[pl.ds(i * MISC + _r * 16, 16)]
        miscred_v[pl.ds(r * 16, 16)] = lax.fori_loop(0, NS, redm, zf)

    wsum_vec = miscred_v[pl.ds(0, 16)]
    for uid in range(1, 9):
        wsum = wsum_vec[uid]
        centers_v[pl.ds(uid * 16, 16)] = (
            centers_v[pl.ds(uid * 16, 16)] / (jnp.full((16,), wsum) + 1e-8))
    centers_v[pl.ds(0, 16)] = zf

    @pl.when(s == 0)
    def _():
        pltpu.sync_copy(centers_v, centers_out.at[pl.ds(b * SUMS, SUMS)])
        pltpu.sync_copy(miscred_v, miscred_out.at[pl.ds(b * MISC, MISC)])

    pull_acc[pl.ds(0, 16)] = zf
    for k in range(ZPW):
        z = z0 + k
        cps = [pltpu.async_copy(
                   embed.at[pl.ds(((b * E + e) * Z + z) * PLANE, PLANE)],
                   emb.at[pl.ds(e * PLANE, PLANE)], sem)
               for e in range(E)]
        pltpu.sync_copy(label.at[pl.ds((b * Z + z) * PLANE, PLANE)], lab_buf)
        pltpu.sync_copy(w_scr.at[pl.ds((b * Z + z) * PLANE, PLANE)], wbuf)
        for cp in cps:
            cp.wait()

        def pchunk(i, _):
            o = i * 16
            labv = lab_buf[pl.ds(o, 16)]
            wv = wbuf[pl.ds(o, 16)]
            lb16 = labv * 16
            d2 = zf
            for e in range(E):
                v = emb[pl.ds(e * PLANE + o, 16)]
                cs = plsc.load_gather(centers_v, [lb16 + e])
                d = v - cs
                d2 = d2 + d * d
            dist = _nsqrt(d2)
            t = jnp.maximum(dist - D_V, 0.0)
            ww = jnp.where(labv == 0, 0.0, wv)
            plsc.addupdate_scatter(pull_acc, [labv], t * t * ww)
            return 0
        lax.fori_loop(0, NCHUNK, pchunk, 0)

    pltpu.sync_copy(pull_acc, pull_out.at[pl.ds(wid * 16, 16)])


@functools.partial(
    pl.kernel,
    out_type=[jax.ShapeDtypeStruct((16,), jnp.float32)],
    mesh=_mesh(),
    compiler_params=pltpu.CompilerParams(needs_layout_passes=False),
    scratch_types=[
        pltpu.VMEM((NW * 16,), jnp.float32),
        pltpu.VMEM((B * SUMS,), jnp.float32),
        pltpu.VMEM((B * MISC,), jnp.float32),
        pltpu.VMEM((16,), jnp.float32),
        pltpu.SemaphoreType.DMA,
    ],
)
def _kern_c(pull_p, centers, miscred, out, pb, cb, mb, ob, sem):
    c = lax.axis_index("c")
    s = lax.axis_index("s")

    @pl.when((c == 0) & (s == 0))
    def _():
        pltpu.sync_copy(pull_p, pb)
        pltpu.sync_copy(centers, cb)
        pltpu.sync_copy(miscred, mb)
        io = lax.iota(jnp.int32, 16)
        uidmask = (io >= 1) & (io <= 8)
        loss_pull = jnp.float32(0.0)
        loss_push = jnp.float32(0.0)
        loss_norm = jnp.float32(0.0)
        for b in range(B):
            def red(i, acc, _b=b):
                return acc + pb[pl.ds((_b * NS + i) * 16, 16)]
            pull_vec = lax.fori_loop(0, NS, red, jnp.zeros((16,), jnp.float32))
            cnt_vec = mb[pl.ds(b * MISC + 16, 16)]
            q = jnp.where(uidmask, pull_vec / cnt_vec, 0.0)
            loss_pull = loss_pull + jnp.sum(q)
            for i in range(1, 9):
                for j in range(i + 1, 9):
                    d = (cb[pl.ds(b * SUMS + i * 16, 16)]
                         - cb[pl.ds(b * SUMS + j * 16, 16)])
                    dd = jnp.max(_nsqrt(jnp.full((16,), jnp.sum(d * d))))
                    pp = jnp.maximum(2.0 * D_D - dd, 0.0)
                    loss_push = loss_push + pp * pp * (1.0 / 28.0)
            for i in range(1, 9):
                cv = cb[pl.ds(b * SUMS + i * 16, 16)]
                loss_norm = loss_norm + jnp.max(
                    _nsqrt(jnp.full((16,), jnp.sum(cv * cv)))) * 0.125
        total = (W_PULL * loss_pull + W_PUSH * loss_push
                 + W_NORM * loss_norm) * (1.0 / B)
        ob[pl.ds(0, 16)] = jnp.full((16,), total)
        pltpu.sync_copy(ob, out)


def kernel(embed, label):
    embed_f = embed.reshape(-1)
    label_f = label.reshape(-1)
    sums_p, misc_p, w_scr = _kern_a(embed_f, label_f)
    pull_p, centers, miscred = _kern_b(embed_f, label_f, w_scr, sums_p, misc_p)
    (out,) = _kern_c(pull_p, centers, miscred)
    return out[0]


# speedup vs baseline: 1.2738x; 1.0124x over previous
import functools

import jax
import jax.numpy as jnp
from jax import lax
from jax.experimental import pallas as pl
from jax.experimental.pallas import tpu as pltpu
from jax.experimental.pallas import tpu_sc as plsc

W_PULL = 1.0
W_PUSH = 1.0
W_NORM = 0.001
W_EDGE = 10.0
D_V = 0.5
D_D = 1.5

B = 2
E = 16
Z = 64
PLANE = 64 * 64
HPLANE = PLANE // 2
NC = 2
NS = 16
NW = NC * NS
ZPW = Z // NS
LPAD = 64
SUMS = 9 * 16
MISC = 2 * 16


def _nsqrt(x):
    i = plsc.bitcast(x, jnp.int32)
    i = jnp.int32(0x5F3759DF) - lax.shift_right_logical(i, 1)
    z = plsc.bitcast(i, jnp.float32)
    z = z * (1.5 - 0.5 * x * z * z)
    z = z * (1.5 - 0.5 * x * z * z)
    z = z * (1.5 - 0.5 * x * z * z)
    return x * z


def _mesh():
    return plsc.VectorSubcoreMesh(
        core_axis_name="c", subcore_axis_name="s", num_cores=NC, num_subcores=NS)


@functools.partial(
    pl.kernel,
    out_type=[
        jax.ShapeDtypeStruct((NW * SUMS,), jnp.float32),
        jax.ShapeDtypeStruct((NW * MISC,), jnp.float32),
        jax.ShapeDtypeStruct((B * Z * PLANE,), jnp.float32),
    ],
    mesh=_mesh(),
    compiler_params=pltpu.CompilerParams(needs_layout_passes=False),
    scratch_types=[
        pltpu.VMEM((PLANE + 2 * LPAD,), jnp.int32),
        pltpu.VMEM((PLANE + 2 * LPAD,), jnp.int32),
        pltpu.VMEM((PLANE + 2 * LPAD,), jnp.int32),
        pltpu.VMEM((PLANE,), jnp.int32),
        pltpu.VMEM((PLANE,), jnp.int32),
        pltpu.VMEM((PLANE,), jnp.int32),
        pltpu.VMEM((PLANE,), jnp.int32),
        pltpu.VMEM((PLANE,), jnp.int32),
        pltpu.VMEM((PLANE,), jnp.int32),
        pltpu.VMEM((PLANE,), jnp.int32),
        pltpu.VMEM((PLANE,), jnp.int32),
        pltpu.VMEM((PLANE,), jnp.float32),
        pltpu.VMEM((E * PLANE,), jnp.float32),
        pltpu.VMEM((SUMS,), jnp.float32),
        pltpu.VMEM((MISC,), jnp.float32),
        pltpu.SemaphoreType.DMA,
    ],
)
def _kern_a(embed, label, sums_out, misc_out, w_out,
            lab0, lab1, lab2, mn0, mn1, mn2, mx0, mx1, mx2, mnx, mxx,
            wbuf, emb, sums_acc, misc_acc, sem):
    labs = [lab0, lab1, lab2]
    mns = [mn0, mn1, mn2]
    mxs = [mx0, mx1, mx2]
    c = lax.axis_index("c")
    s = lax.axis_index("s")
    wid = c * NS + s
    b = c
    z0 = s * ZPW
    io = lax.iota(jnp.int32, 16)
    zf = jnp.zeros((16,), jnp.float32)
    onef = jnp.ones((16,), jnp.float32)

    for r in range(SUMS // 16):
        sums_acc[pl.ds(r * 16, 16)] = zf
    misc_acc[pl.ds(0, 16)] = zf
    misc_acc[pl.ds(16, 16)] = zf

    def load_lab(z, slot):
        zc = jnp.clip(z, 0, Z - 1)
        pltpu.sync_copy(label.at[pl.ds((b * Z + zc) * PLANE, PLANE)],
                        labs[slot].at[pl.ds(LPAD, PLANE)])

    def xy_pass(slot):
        lab = labs[slot]
        mnr = mns[slot]
        mxr = mxs[slot]

        def xrow(r, _):
            base = LPAD + r * 64
            for p in range(4):
                o = base + p * 16
                cv = lab[pl.ds(o, 16)]
                lv = lab[pl.ds(o - 1, 16)]
                rv = lab[pl.ds(o + 1, 16)]
                if p == 0:
                    lv = jnp.where(io == 0, cv, lv)
                if p == 3:
                    rv = jnp.where(io == 15, cv, rv)
                oo = r * 64 + p * 16
                mnx[pl.ds(oo, 16)] = jnp.minimum(jnp.minimum(lv, cv), rv)
                mxx[pl.ds(oo, 16)] = jnp.maximum(jnp.maximum(lv, cv), rv)
            return 0
        lax.fori_loop(0, 64, xrow, 0)

        def yrow(r, _):
            rm = jnp.maximum(r - 1, 0) * 64
            rc = r * 64
            rp = jnp.minimum(r + 1, 63) * 64
            for p in range(4):
                q = p * 16
                mnr[pl.ds(rc + q, 16)] = jnp.minimum(
                    jnp.minimum(mnx[pl.ds(rm + q, 16)], mnx[pl.ds(rc + q, 16)]),
                    mnx[pl.ds(rp + q, 16)])
                mxr[pl.ds(rc + q, 16)] = jnp.maximum(
                    jnp.maximum(mxx[pl.ds(rm + q, 16)], mxx[pl.ds(rc + q, 16)]),
                    mxx[pl.ds(rp + q, 16)])
            return 0
        lax.fori_loop(0, 64, yrow, 0)

    load_lab(z0 - 1, 0)
    xy_pass(0)
    load_lab(z0, 1)
    xy_pass(1)

    for k in range(ZPW):
        z = z0 + k
        sl_cur = (k + 1) % 3
        sl_next = (k + 2) % 3

        cps = [pltpu.async_copy(
                   embed.at[pl.ds(((b * E + e) * Z + z) * PLANE, PLANE)],
                   emb.at[pl.ds(e * PLANE, PLANE)], sem)
               for e in range(E)]

        load_lab(z + 1, sl_next)
        xy_pass(sl_next)

        mnp_, mnc_, mnn_ = mns[k % 3], mns[sl_cur], mns[sl_next]
        mxp_, mxc_, mxn_ = mxs[k % 3], mxs[sl_cur], mxs[sl_next]

        def wchunk(i, _):
            for u in range(4):
                o = i * 64 + u * 16
                mnv = jnp.minimum(
                    jnp.minimum(mnp_[pl.ds(o, 16)], mnc_[pl.ds(o, 16)]),
                    mnn_[pl.ds(o, 16)])
                mxv = jnp.maximum(
                    jnp.maximum(mxp_[pl.ds(o, 16)], mxc_[pl.ds(o, 16)]),
                    mxn_[pl.ds(o, 16)])
                wbuf[pl.ds(o, 16)] = jnp.where(mxv != mnv, W_EDGE, 1.0)
            return 0
        lax.fori_loop(0, PLANE // 64, wchunk, 0)

        pltpu.sync_copy(wbuf, w_out.at[pl.ds((b * Z + z) * PLANE, PLANE)])
        for cp in cps:
            cp.wait()

        labc_ = labs[sl_cur]

        def achunk(i, _):
            for u in range(4):
                o = i * 64 + u * 16
                labv = labc_[pl.ds(LPAD + o, 16)]
                wv = wbuf[pl.ds(o, 16)]
                plsc.addupdate_scatter(misc_acc, [labv], wv)
                plsc.addupdate_scatter(misc_acc, [labv + 16], onef)
                lb16 = labv * 16
                for e in range(E):
                    v = emb[pl.ds(e * PLANE + o, 16)]
                    plsc.addupdate_scatter(sums_acc, [lb16 + e], wv * v)
            return 0
        lax.fori_loop(0, PLANE // 64, achunk, 0)

    pltpu.sync_copy(sums_acc, sums_out.at[pl.ds(wid * SUMS, SUMS)])
    pltpu.sync_copy(misc_acc, misc_out.at[pl.ds(wid * MISC, MISC)])


@functools.partial(
    pl.kernel,
    out_type=[
        jax.ShapeDtypeStruct((NW * 16,), jnp.float32),
        jax.ShapeDtypeStruct((B * SUMS,), jnp.float32),
        jax.ShapeDtypeStruct((B * MISC,), jnp.float32),
    ],
    mesh=_mesh(),
    compiler_params=pltpu.CompilerParams(needs_layout_passes=False),
    scratch_types=[
        pltpu.VMEM((NS * SUMS,), jnp.float32),
        pltpu.VMEM((NS * MISC,), jnp.float32),
        pltpu.VMEM((SUMS,), jnp.float32),
        pltpu.VMEM((MISC,), jnp.float32),
        pltpu.VMEM((PLANE,), jnp.int32),
        pltpu.VMEM((PLANE,), jnp.float32),
        pltpu.VMEM((E * HPLANE,), jnp.float32),
        pltpu.VMEM((E * HPLANE,), jnp.float32),
        pltpu.VMEM((16,), jnp.float32),
        pltpu.SemaphoreType.DMA,
        pltpu.SemaphoreType.DMA,
    ],
)
def _kern_b(embed, label, w_scr, sums_p, misc_p,
            pull_out, centers_out, miscred_out,
            part_buf, mpart_buf, centers_v, miscred_v, lab_buf, wbuf,
            emb0, emb1, pull_acc, sem0, sem1):
    embs = [emb0, emb1]
    sems = [sem0, sem1]
    c = lax.axis_index("c")
    s = lax.axis_index("s")
    wid = c * NS + s
    b = c
    z0 = s * ZPW
    zf = jnp.zeros((16,), jnp.float32)

    def fire(t):
        k, h = t // 2, t % 2
        z = z0 + k
        buf = embs[t % 2]
        return [pltpu.async_copy(
                    embed.at[pl.ds(((b * E + e) * Z + z) * PLANE + h * HPLANE,
                                   HPLANE)],
                    buf.at[pl.ds(e * HPLANE, HPLANE)], sems[t % 2])
                for e in range(E)]

    cps = fire(0)

    pltpu.sync_copy(sums_p.at[pl.ds(b * NS * SUMS, NS * SUMS)], part_buf)
    pltpu.sync_copy(misc_p.at[pl.ds(b * NS * MISC, NS * MISC)], mpart_buf)

    for uid in range(9):
        def red(i, acc, _uid=uid):
            return acc + part_buf[pl.ds(i * SUMS + _uid * 16, 16)]
        centers_v[pl.ds(uid * 16, 16)] = lax.fori_loop(0, NS, red, zf)
    for r in range(2):
        def redm(i, acc, _r=r):
            return acc + mpart_buf[pl.ds(i * MISC + _r * 16, 16)]
        miscred_v[pl.ds(r * 16, 16)] = lax.fori_loop(0, NS, redm, zf)

    wsum_vec = miscred_v[pl.ds(0, 16)]
    for uid in range(1, 9):
        wsum = wsum_vec[uid]
        centers_v[pl.ds(uid * 16, 16)] = (
            centers_v[pl.ds(uid * 16, 16)] / (jnp.full((16,), wsum) + 1e-8))
    centers_v[pl.ds(0, 16)] = zf

    @pl.when(s == 0)
    def _():
        pltpu.sync_copy(centers_v, centers_out.at[pl.ds(b * SUMS, SUMS)])
        pltpu.sync_copy(miscred_v, miscred_out.at[pl.ds(b * MISC, MISC)])

    pull_acc[pl.ds(0, 16)] = zf
    for t in range(2 * ZPW):
        k, h = t // 2, t % 2
        z = z0 + k
        if h == 0:
            pltpu.sync_copy(label.at[pl.ds((b * Z + z) * PLANE, PLANE)],
                            lab_buf)
            pltpu.sync_copy(w_scr.at[pl.ds((b * Z + z) * PLANE, PLANE)], wbuf)
        for cp in cps:
            cp.wait()
        if t + 1 < 2 * ZPW:
            cps = fire(t + 1)
        embx = embs[t % 2]
        hb = h * HPLANE

        def pchunk(i, _):
            for u in range(2):
                o = i * 32 + u * 16
                labv = lab_buf[pl.ds(hb + o, 16)]
                wv = wbuf[pl.ds(hb + o, 16)]
                lb16 = labv * 16
                sq = []
                for e in range(E):
                    v = embx[pl.ds(e * HPLANE + o, 16)]
                    cs = plsc.load_gather(centers_v, [lb16 + e])
                    d = v - cs
                    sq.append(d * d)
                while len(sq) > 1:
                    sq = [a + bq for a, bq in zip(sq[::2], sq[1::2])]
                dist = _nsqrt(sq[0])
                t_ = jnp.maximum(dist - D_V, 0.0)
                ww = jnp.where(labv == 0, 0.0, wv)
                plsc.addupdate_scatter(pull_acc, [labv], t_ * t_ * ww)
            return 0
        lax.fori_loop(0, HPLANE // 32, pchunk, 0)

    pltpu.sync_copy(pull_acc, pull_out.at[pl.ds(wid * 16, 16)])


@functools.partial(
    pl.kernel,
    out_type=[jax.ShapeDtypeStruct((16,), jnp.float32)],
    mesh=_mesh(),
    compiler_params=pltpu.CompilerParams(needs_layout_passes=False),
    scratch_types=[
        pltpu.VMEM((NW * 16,), jnp.float32),
        pltpu.VMEM((B * SUMS,), jnp.float32),
        pltpu.VMEM((B * MISC,), jnp.float32),
        pltpu.VMEM((16,), jnp.float32),
        pltpu.SemaphoreType.DMA,
    ],
)
def _kern_c(pull_p, centers, miscred, out, pb, cb, mb, ob, sem):
    c = lax.axis_index("c")
    s = lax.axis_index("s")

    @pl.when((c == 0) & (s == 0))
    def _():
        pltpu.sync_copy(pull_p, pb)
        pltpu.sync_copy(centers, cb)
        pltpu.sync_copy(miscred, mb)
        io = lax.iota(jnp.int32, 16)
        uidmask = (io >= 1) & (io <= 8)
        loss_pull = jnp.float32(0.0)
        loss_push = jnp.float32(0.0)
        loss_norm = jnp.float32(0.0)
        for b in range(B):
            def red(i, acc, _b=b):
                return acc + pb[pl.ds((_b * NS + i) * 16, 16)]
            pull_vec = lax.fori_loop(0, NS, red, jnp.zeros((16,), jnp.float32))
            cnt_vec = mb[pl.ds(b * MISC + 16, 16)]
            q = jnp.where(uidmask, pull_vec / cnt_vec, 0.0)
            loss_pull = loss_pull + jnp.sum(q)
            for i in range(1, 9):
                for j in range(i + 1, 9):
                    d = (cb[pl.ds(b * SUMS + i * 16, 16)]
                         - cb[pl.ds(b * SUMS + j * 16, 16)])
                    dd = jnp.max(_nsqrt(jnp.full((16,), jnp.sum(d * d))))
                    pp = jnp.maximum(2.0 * D_D - dd, 0.0)
                    loss_push = loss_push + pp * pp * (1.0 / 28.0)
            for i in range(1, 9):
                cv = cb[pl.ds(b * SUMS + i * 16, 16)]
                loss_norm = loss_norm + jnp.max(
                    _nsqrt(jnp.full((16,), jnp.sum(cv * cv)))) * 0.125
        total = (W_PULL * loss_pull + W_PUSH * loss_push
                 + W_NORM * loss_norm) * (1.0 / B)
        ob[pl.ds(0, 16)] = jnp.full((16,), total)
        pltpu.sync_copy(ob, out)


def kernel(embed, label):
    embed_f = embed.reshape(-1)
    label_f = label.reshape(-1)
    sums_p, misc_p, w_scr = _kern_a(embed_f, label_f)
    pull_p, centers, miscred = _kern_b(embed_f, label_f, w_scr, sums_p, misc_p)
    (out,) = _kern_c(pull_p, centers, miscred)
    return out[0]


# speedup vs baseline: 1.6349x; 1.2836x over previous
import functools

import jax
import jax.numpy as jnp
from jax import lax
from jax.experimental import pallas as pl
from jax.experimental.pallas import tpu as pltpu
from jax.experimental.pallas import tpu_sc as plsc

W_PULL = 1.0
W_PUSH = 1.0
W_NORM = 0.001
W_EDGE = 10.0
D_V = 0.5
D_D = 1.5

B = 2
E = 16
Z = 64
PLANE = 64 * 64
HPLANE = PLANE // 2
NC = 2
NS = 16
NW = NC * NS
ZPW = Z // NS
LPAD = 64
SUMS = E * 16
CEN = 9 * 16
MISC = 2 * 16


def _nsqrt(x):
    i = plsc.bitcast(x, jnp.int32)
    i = jnp.int32(0x5F3759DF) - lax.shift_right_logical(i, 1)
    z = plsc.bitcast(i, jnp.float32)
    z = z * (1.5 - 0.5 * x * z * z)
    z = z * (1.5 - 0.5 * x * z * z)
    z = z * (1.5 - 0.5 * x * z * z)
    return x * z


def _mesh():
    return plsc.VectorSubcoreMesh(
        core_axis_name="c", subcore_axis_name="s", num_cores=NC, num_subcores=NS)


@functools.partial(
    pl.kernel,
    out_type=[
        jax.ShapeDtypeStruct((NW * SUMS,), jnp.float32),
        jax.ShapeDtypeStruct((NW * MISC,), jnp.float32),
        jax.ShapeDtypeStruct((B * Z * PLANE,), jnp.float32),
    ],
    mesh=_mesh(),
    compiler_params=pltpu.CompilerParams(needs_layout_passes=False),
    scratch_types=[
        pltpu.VMEM((PLANE + 2 * LPAD,), jnp.int32),
        pltpu.VMEM((PLANE + 2 * LPAD,), jnp.int32),
        pltpu.VMEM((PLANE + 2 * LPAD,), jnp.int32),
        pltpu.VMEM((PLANE,), jnp.int32),
        pltpu.VMEM((PLANE,), jnp.int32),
        pltpu.VMEM((PLANE,), jnp.int32),
        pltpu.VMEM((PLANE,), jnp.int32),
        pltpu.VMEM((PLANE,), jnp.int32),
        pltpu.VMEM((PLANE,), jnp.int32),
        pltpu.VMEM((PLANE,), jnp.int32),
        pltpu.VMEM((PLANE,), jnp.int32),
        pltpu.VMEM((PLANE,), jnp.float32),
        pltpu.VMEM((E * PLANE,), jnp.float32),
        pltpu.VMEM((SUMS,), jnp.float32),
        pltpu.VMEM((MISC,), jnp.float32),
        pltpu.SemaphoreType.DMA,
    ],
)
def _kern_a(embed, label, sums_out, misc_out, w_out,
            lab0, lab1, lab2, mn0, mn1, mn2, mx0, mx1, mx2, mnx, mxx,
            wbuf, emb, sums_acc, misc_acc, sem):
    labs = [lab0, lab1, lab2]
    mns = [mn0, mn1, mn2]
    mxs = [mx0, mx1, mx2]
    c = lax.axis_index("c")
    s = lax.axis_index("s")
    wid = c * NS + s
    b = c
    z0 = s * ZPW
    io = lax.iota(jnp.int32, 16)
    zf = jnp.zeros((16,), jnp.float32)
    onef = jnp.ones((16,), jnp.float32)

    for r in range(SUMS // 16):
        sums_acc[pl.ds(r * 16, 16)] = zf
    misc_acc[pl.ds(0, 16)] = zf
    misc_acc[pl.ds(16, 16)] = zf

    def load_lab(z, slot):
        zc = jnp.clip(z, 0, Z - 1)
        pltpu.sync_copy(label.at[pl.ds((b * Z + zc) * PLANE, PLANE)],
                        labs[slot].at[pl.ds(LPAD, PLANE)])

    def xy_pass(slot):
        lab = labs[slot]
        mnr = mns[slot]
        mxr = mxs[slot]

        def xrow(r, _):
            base = LPAD + r * 64
            for p in range(4):
                o = base + p * 16
                cv = lab[pl.ds(o, 16)]
                lv = lab[pl.ds(o - 1, 16)]
                rv = lab[pl.ds(o + 1, 16)]
                if p == 0:
                    lv = jnp.where(io == 0, cv, lv)
                if p == 3:
                    rv = jnp.where(io == 15, cv, rv)
                oo = r * 64 + p * 16
                mnx[pl.ds(oo, 16)] = jnp.minimum(jnp.minimum(lv, cv), rv)
                mxx[pl.ds(oo, 16)] = jnp.maximum(jnp.maximum(lv, cv), rv)
            return 0
        lax.fori_loop(0, 64, xrow, 0)

        def yrow(r, _):
            rm = jnp.maximum(r - 1, 0) * 64
            rc = r * 64
            rp = jnp.minimum(r + 1, 63) * 64
            for p in range(4):
                q = p * 16
                mnr[pl.ds(rc + q, 16)] = jnp.minimum(
                    jnp.minimum(mnx[pl.ds(rm + q, 16)], mnx[pl.ds(rc + q, 16)]),
                    mnx[pl.ds(rp + q, 16)])
                mxr[pl.ds(rc + q, 16)] = jnp.maximum(
                    jnp.maximum(mxx[pl.ds(rm + q, 16)], mxx[pl.ds(rc + q, 16)]),
                    mxx[pl.ds(rp + q, 16)])
            return 0
        lax.fori_loop(0, 64, yrow, 0)

    load_lab(z0 - 1, 0)
    xy_pass(0)
    load_lab(z0, 1)
    xy_pass(1)

    for k in range(ZPW):
        z = z0 + k
        sl_cur = (k + 1) % 3
        sl_next = (k + 2) % 3

        cps = [pltpu.async_copy(
                   embed.at[pl.ds(((b * E + e) * Z + z) * PLANE, PLANE)],
                   emb.at[pl.ds(e * PLANE, PLANE)], sem)
               for e in range(E)]

        load_lab(z + 1, sl_next)
        xy_pass(sl_next)

        mnp_, mnc_, mnn_ = mns[k % 3], mns[sl_cur], mns[sl_next]
        mxp_, mxc_, mxn_ = mxs[k % 3], mxs[sl_cur], mxs[sl_next]

        def wchunk(i, _):
            for u in range(4):
                o = i * 64 + u * 16
                mnv = jnp.minimum(
                    jnp.minimum(mnp_[pl.ds(o, 16)], mnc_[pl.ds(o, 16)]),
                    mnn_[pl.ds(o, 16)])
                mxv = jnp.maximum(
                    jnp.maximum(mxp_[pl.ds(o, 16)], mxc_[pl.ds(o, 16)]),
                    mxn_[pl.ds(o, 16)])
                wbuf[pl.ds(o, 16)] = jnp.where(mxv != mnv, W_EDGE, 1.0)
            return 0
        lax.fori_loop(0, PLANE // 64, wchunk, 0)

        pltpu.sync_copy(wbuf, w_out.at[pl.ds((b * Z + z) * PLANE, PLANE)])
        for cp in cps:
            cp.wait()

        labc_ = labs[sl_cur]

        def achunk(i, _):
            for u in range(4):
                o = i * 64 + u * 16
                labv = labc_[pl.ds(LPAD + o, 16)]
                wv = wbuf[pl.ds(o, 16)]
                plsc.addupdate_scatter(misc_acc, [labv], wv)
                plsc.addupdate_scatter(misc_acc, [labv + 16], onef)
                for e in range(E):
                    v = emb[pl.ds(e * PLANE + o, 16)]
                    plsc.addupdate_scatter(sums_acc, [labv + e * 16], wv * v)
            return 0
        lax.fori_loop(0, PLANE // 64, achunk, 0)

    pltpu.sync_copy(sums_acc, sums_out.at[pl.ds(wid * SUMS, SUMS)])
    pltpu.sync_copy(misc_acc, misc_out.at[pl.ds(wid * MISC, MISC)])


@functools.partial(
    pl.kernel,
    out_type=[
        jax.ShapeDtypeStruct((NW * 16,), jnp.float32),
        jax.ShapeDtypeStruct((B * CEN,), jnp.float32),
        jax.ShapeDtypeStruct((B * MISC,), jnp.float32),
    ],
    mesh=_mesh(),
    compiler_params=pltpu.CompilerParams(needs_layout_passes=False),
    scratch_types=[
        pltpu.VMEM((NS * SUMS,), jnp.float32),
        pltpu.VMEM((NS * MISC,), jnp.float32),
        pltpu.VMEM((SUMS,), jnp.float32),
        pltpu.VMEM((CEN,), jnp.float32),
        pltpu.VMEM((MISC,), jnp.float32),
        pltpu.VMEM((PLANE,), jnp.int32),
        pltpu.VMEM((PLANE,), jnp.float32),
        pltpu.VMEM((E * HPLANE,), jnp.float32),
        pltpu.VMEM((E * HPLANE,), jnp.float32),
        pltpu.VMEM((16,), jnp.float32),
        pltpu.SemaphoreType.DMA,
        pltpu.SemaphoreType.DMA,
    ],
)
def _kern_b(embed, label, w_scr, sums_p, misc_p,
            pull_out, centers_out, miscred_out,
            part_buf, mpart_buf, centers_v, cent_t, miscred_v, lab_buf, wbuf,
            emb0, emb1, pull_acc, sem0, sem1):
    embs = [emb0, emb1]
    sems = [sem0, sem1]
    c = lax.axis_index("c")
    s = lax.axis_index("s")
    wid = c * NS + s
    b = c
    z0 = s * ZPW
    zf = jnp.zeros((16,), jnp.float32)

    def fire(t):
        k, h = t // 2, t % 2
        z = z0 + k
        buf = embs[t % 2]
        return [pltpu.async_copy(
                    embed.at[pl.ds(((b * E + e) * Z + z) * PLANE + h * HPLANE,
                                   HPLANE)],
                    buf.at[pl.ds(e * HPLANE, HPLANE)], sems[t % 2])
                for e in range(E)]

    cps = fire(0)

    pltpu.sync_copy(sums_p.at[pl.ds(b * NS * SUMS, NS * SUMS)], part_buf)
    pltpu.sync_copy(misc_p.at[pl.ds(b * NS * MISC, NS * MISC)], mpart_buf)

    for r in range(2):
        def redm(i, acc, _r=r):
            return acc + mpart_buf[pl.ds(i * MISC + _r * 16, 16)]
        miscred_v[pl.ds(r * 16, 16)] = lax.fori_loop(0, NS, redm, zf)

    wsum_vec = miscred_v[pl.ds(0, 16)] + 1e-8
    for e in range(E):
        def red(i, acc, _e=e):
            return acc + part_buf[pl.ds(i * SUMS + _e * 16, 16)]
        centers_v[pl.ds(e * 16, 16)] = (
            lax.fori_loop(0, NS, red, zf) / wsum_vec)

    @pl.when(s == 0)
    def _():
        io = lax.iota(jnp.int32, 16)
        cent_t[pl.ds(0, 16)] = zf
        for uid in range(1, 9):
            cent_t[pl.ds(uid * 16, 16)] = plsc.load_gather(
                centers_v, [io * 16 + uid])
        pltpu.sync_copy(cent_t, centers_out.at[pl.ds(b * CEN, CEN)])
        pltpu.sync_copy(miscred_v, miscred_out.at[pl.ds(b * MISC, MISC)])

    pull_acc[pl.ds(0, 16)] = zf
    for t in range(2 * ZPW):
        k, h = t // 2, t % 2
        z = z0 + k
        if h == 0:
            pltpu.sync_copy(label.at[pl.ds((b * Z + z) * PLANE, PLANE)],
                            lab_buf)
            pltpu.sync_copy(w_scr.at[pl.ds((b * Z + z) * PLANE, PLANE)], wbuf)
        for cp in cps:
            cp.wait()
        if t + 1 < 2 * ZPW:
            cps = fire(t + 1)
        embx = embs[t % 2]
        hb = h * HPLANE

        def pchunk(i, _):
            for u in range(2):
                o = i * 32 + u * 16
                labv = lab_buf[pl.ds(hb + o, 16)]
                wv = wbuf[pl.ds(hb + o, 16)]
                sq = []
                for e in range(E):
                    v = embx[pl.ds(e * HPLANE + o, 16)]
                    cs = plsc.load_gather(centers_v, [labv + e * 16])
                    d = v - cs
                    sq.append(d * d)
                while len(sq) > 1:
                    sq = [a + bq for a, bq in zip(sq[::2], sq[1::2])]
                dist = _nsqrt(sq[0])
                t_ = jnp.maximum(dist - D_V, 0.0)
                ww = jnp.where(labv == 0, 0.0, wv)
                plsc.addupdate_scatter(pull_acc, [labv], t_ * t_ * ww)
            return 0
        lax.fori_loop(0, HPLANE // 32, pchunk, 0)

    pltpu.sync_copy(pull_acc, pull_out.at[pl.ds(wid * 16, 16)])


@functools.partial(
    pl.kernel,
    out_type=[jax.ShapeDtypeStruct((16,), jnp.float32)],
    mesh=_mesh(),
    compiler_params=pltpu.CompilerParams(needs_layout_passes=False),
    scratch_types=[
        pltpu.VMEM((NW * 16,), jnp.float32),
        pltpu.VMEM((B * CEN,), jnp.float32),
        pltpu.VMEM((B * MISC,), jnp.float32),
        pltpu.VMEM((16,), jnp.float32),
        pltpu.SemaphoreType.DMA,
    ],
)
def _kern_c(pull_p, centers, miscred, out, pb, cb, mb, ob, sem):
    c = lax.axis_index("c")
    s = lax.axis_index("s")

    @pl.when((c == 0) & (s == 0))
    def _():
        pltpu.sync_copy(pull_p, pb)
        pltpu.sync_copy(centers, cb)
        pltpu.sync_copy(miscred, mb)
        io = lax.iota(jnp.int32, 16)
        uidmask = (io >= 1) & (io <= 8)
        loss_pull = jnp.float32(0.0)
        loss_push = jnp.float32(0.0)
        loss_norm = jnp.float32(0.0)
        for b in range(B):
            def red(i, acc, _b=b):
                return acc + pb[pl.ds((_b * NS + i) * 16, 16)]
            pull_vec = lax.fori_loop(0, NS, red, jnp.zeros((16,), jnp.float32))
            cnt_vec = mb[pl.ds(b * MISC + 16, 16)]
            q = jnp.where(uidmask, pull_vec / cnt_vec, 0.0)
            loss_pull = loss_pull + jnp.sum(q)
            for i in range(1, 9):
                for j in range(i + 1, 9):
                    d = (cb[pl.ds(b * CEN + i * 16, 16)]
                         - cb[pl.ds(b * CEN + j * 16, 16)])
                    dd = jnp.max(_nsqrt(jnp.full((16,), jnp.sum(d * d))))
                    pp = jnp.maximum(2.0 * D_D - dd, 0.0)
                    loss_push = loss_push + pp * pp * (1.0 / 28.0)
            for i in range(1, 9):
                cv = cb[pl.ds(b * CEN + i * 16, 16)]
                loss_norm = loss_norm + jnp.max(
                    _nsqrt(jnp.full((16,), jnp.sum(cv * cv)))) * 0.125
        total = (W_PULL * loss_pull + W_PUSH * loss_push
                 + W_NORM * loss_norm) * (1.0 / B)
        ob[pl.ds(0, 16)] = jnp.full((16,), total)
        pltpu.sync_copy(ob, out)


def kernel(embed, label):
    embed_f = embed.reshape(-1)
    label_f = label.reshape(-1)
    sums_p, misc_p, w_scr = _kern_a(embed_f, label_f)
    pull_p, centers, miscred = _kern_b(embed_f, label_f, w_scr, sums_p, misc_p)
    (out,) = _kern_c(pull_p, centers, miscred)
    return out[0]


# speedup vs baseline: 1.9765x; 1.2089x over previous
import functools

import jax
import jax.numpy as jnp
from jax import lax
from jax.experimental import pallas as pl
from jax.experimental.pallas import tpu as pltpu
from jax.experimental.pallas import tpu_sc as plsc

W_PULL = 1.0
W_PUSH = 1.0
W_NORM = 0.001
W_EDGE = 10.0
D_V = 0.5
D_D = 1.5

B = 2
E = 16
Z = 64
PLANE = 64 * 64
HPLANE = PLANE // 2
NC = 2
NS = 16
NW = NC * NS
ZPW = Z // NS
NSEG = 2 * ZPW
LPAD = 64
PART = 18 * 16
CEN = 9 * 16
MISC = 2 * 16


def _nsqrt(x):
    i = plsc.bitcast(x, jnp.int32)
    i = jnp.int32(0x5F3759DF) - lax.shift_right_logical(i, 1)
    z = plsc.bitcast(i, jnp.float32)
    z = z * (1.5 - 0.5 * x * z * z)
    z = z * (1.5 - 0.5 * x * z * z)
    z = z * (1.5 - 0.5 * x * z * z)
    return x * z


def _mesh():
    return plsc.VectorSubcoreMesh(
        core_axis_name="c", subcore_axis_name="s", num_cores=NC, num_subcores=NS)


@functools.partial(
    pl.kernel,
    out_type=[
        jax.ShapeDtypeStruct((NW * 16,), jnp.float32),
        jax.ShapeDtypeStruct((B * CEN,), jnp.float32),
        jax.ShapeDtypeStruct((B * MISC,), jnp.float32),
        jax.ShapeDtypeStruct((B * Z * PLANE,), jnp.float32),
    ],
    mesh=_mesh(),
    compiler_params=pltpu.CompilerParams(needs_layout_passes=False),
    scratch_types=[
        pltpu.VMEM((PLANE + 2 * LPAD,), jnp.int32),
        pltpu.VMEM((PLANE + 2 * LPAD,), jnp.int32),
        pltpu.VMEM((PLANE + 2 * LPAD,), jnp.int32),
        pltpu.VMEM((PLANE,), jnp.int32),
        pltpu.VMEM((PLANE,), jnp.int32),
        pltpu.VMEM((PLANE,), jnp.int32),
        pltpu.VMEM((PLANE,), jnp.int32),
        pltpu.VMEM((PLANE,), jnp.int32),
        pltpu.VMEM((PLANE,), jnp.int32),
        pltpu.VMEM((PLANE,), jnp.int32),
        pltpu.VMEM((PLANE,), jnp.int32),
        pltpu.VMEM((PLANE,), jnp.float32),
        pltpu.VMEM((E * HPLANE,), jnp.float32),
        pltpu.VMEM((E * HPLANE,), jnp.float32),
        pltpu.VMEM((16 * PART,), jnp.float32),
        pltpu.VMEM((PART,), jnp.float32),
        pltpu.VMEM((E * 16,), jnp.float32),
        pltpu.VMEM((16 * E * 16,), jnp.float32),
        pltpu.VMEM((CEN,), jnp.float32),
        pltpu.VMEM_SHARED((NS * PART,), jnp.float32),
        pltpu.SemaphoreType.DMA,
        pltpu.SemaphoreType.DMA,
    ],
)
def _kern_ab(embed, label, pull_out, centers_out, miscred_out, w_out,
             lab0, lab1, lab2, mn0, mn1, mn2, mx0, mx1, mx2, mnx, mxx,
             wbuf, emb0, emb1, rep, part_v, centers_v, crep, cent_t,
             shared, sem0, sem1):
    labs = [lab0, lab1, lab2]
    mns = [mn0, mn1, mn2]
    mxs = [mx0, mx1, mx2]
    embs = [emb0, emb1]
    sems = [sem0, sem1]
    c = lax.axis_index("c")
    s = lax.axis_index("s")
    wid = c * NS + s
    b = c
    z0 = s * ZPW
    io = lax.iota(jnp.int32, 16)
    zf = jnp.zeros((16,), jnp.float32)
    onef = jnp.ones((16,), jnp.float32)

    def zrow(r, _):
        rep[pl.ds(r * 16, 16)] = zf
        return 0
    lax.fori_loop(0, 16 * PART // 16, zrow, 0)

    def fire(t):
        k, h = t // 2, t % 2
        z = z0 + k
        buf = embs[t % 2]
        return [pltpu.async_copy(
                    embed.at[pl.ds(((b * E + e) * Z + z) * PLANE + h * HPLANE,
                                   HPLANE)],
                    buf.at[pl.ds(e * HPLANE, HPLANE)], sems[t % 2])
                for e in range(E)]

    cps = fire(0)

    def load_lab(z, slot):
        zc = jnp.clip(z, 0, Z - 1)
        pltpu.sync_copy(label.at[pl.ds((b * Z + zc) * PLANE, PLANE)],
                        labs[slot].at[pl.ds(LPAD, PLANE)])

    def xy_pass(slot):
        lab = labs[slot]
        mnr = mns[slot]
        mxr = mxs[slot]

        def xrow(r, _):
            base = LPAD + r * 64
            for p in range(4):
                o = base + p * 16
                cv = lab[pl.ds(o, 16)]
                lv = lab[pl.ds(o - 1, 16)]
                rv = lab[pl.ds(o + 1, 16)]
                if p == 0:
                    lv = jnp.where(io == 0, cv, lv)
                if p == 3:
                    rv = jnp.where(io == 15, cv, rv)
                oo = r * 64 + p * 16
                mnx[pl.ds(oo, 16)] = jnp.minimum(jnp.minimum(lv, cv), rv)
                mxx[pl.ds(oo, 16)] = jnp.maximum(jnp.maximum(lv, cv), rv)
            return 0
        lax.fori_loop(0, 64, xrow, 0)

        def yrow(r, _):
            rm = jnp.maximum(r - 1, 0) * 64
            rc = r * 64
            rp = jnp.minimum(r + 1, 63) * 64
            for p in range(4):
                q = p * 16
                mnr[pl.ds(rc + q, 16)] = jnp.minimum(
                    jnp.minimum(mnx[pl.ds(rm + q, 16)], mnx[pl.ds(rc + q, 16)]),
                    mnx[pl.ds(rp + q, 16)])
                mxr[pl.ds(rc + q, 16)] = jnp.maximum(
                    jnp.maximum(mxx[pl.ds(rm + q, 16)], mxx[pl.ds(rc + q, 16)]),
                    mxx[pl.ds(rp + q, 16)])
            return 0
        lax.fori_loop(0, 64, yrow, 0)

    load_lab(z0 - 1, 0)
    xy_pass(0)
    load_lab(z0, 1)
    xy_pass(1)

    for k in range(ZPW):
        z = z0 + k
        sl_cur = (k + 1) % 3
        sl_next = (k + 2) % 3

        load_lab(z + 1, sl_next)
        xy_pass(sl_next)

        mnp_, mnc_, mnn_ = mns[k % 3], mns[sl_cur], mns[sl_next]
        mxp_, mxc_, mxn_ = mxs[k % 3], mxs[sl_cur], mxs[sl_next]
        labc_ = labs[sl_cur]

        for h in range(2):
            t = 2 * k + h
            for cp in cps:
                cp.wait()
            if t + 1 < NSEG:
                cps = fire(t + 1)
            embx = embs[t % 2]
            hb = h * HPLANE

            def achunk(i, _, _hb=hb, _embx=embx, _mn=(mnp_, mnc_, mnn_),
                       _mx=(mxp_, mxc_, mxn_), _lab=labc_):
                for u in range(1):
                    o = i * 16
                    po = _hb + o
                    mnv = jnp.minimum(
                        jnp.minimum(_mn[0][pl.ds(po, 16)],
                                    _mn[1][pl.ds(po, 16)]),
                        _mn[2][pl.ds(po, 16)])
                    mxv = jnp.maximum(
                        jnp.maximum(_mx[0][pl.ds(po, 16)],
                                    _mx[1][pl.ds(po, 16)]),
                        _mx[2][pl.ds(po, 16)])
                    wv = jnp.where(mxv != mnv, W_EDGE, 1.0)
                    wbuf[pl.ds(po, 16)] = wv
                    labv = _lab[pl.ds(LPAD + po, 16)]
                    base = labv * 16 + io
                    plsc.addupdate_scatter(rep, [base + 16 * 256], wv)
                    plsc.addupdate_scatter(rep, [base + 17 * 256], onef)
                    for e in range(E):
                        v = _embx[pl.ds(e * HPLANE + o, 16)]
                        plsc.addupdate_scatter(rep, [base + e * 256], wv * v)
                return 0
            lax.fori_loop(0, HPLANE // 16, achunk, 0)

        pltpu.sync_copy(wbuf, w_out.at[pl.ds((b * Z + z) * PLANE, PLANE)])

    cps = fire(0)

    def fold(g, _):
        acc = zf
        for l in range(16):
            acc = acc + plsc.load_gather(rep, [g * 256 + io * 16 + l])
        part_v[pl.ds(g * 16, 16)] = acc
        return 0
    lax.fori_loop(0, 18, fold, 0)

    pltpu.sync_copy(part_v, shared.at[pl.ds(s * PART, PART)])
    plsc.subcore_barrier()
    pltpu.sync_copy(shared, rep.at[pl.ds(0, NS * PART)])

    def redrow(g):
        def red(i, acc, _g=g):
            return acc + rep[pl.ds(i * PART + _g * 16, 16)]
        return lax.fori_loop(0, NS, red, zf)

    wsum_vec = redrow(16) + 1e-8
    cnt_vec = redrow(17)
    for e in range(E):
        centers_v[pl.ds(e * 16, 16)] = redrow(e) / wsum_vec

    def crow(e, _):
        row = centers_v[pl.ds(e * 16, 16)]
        for l in range(16):
            plsc.store_scatter(crep, [e * 256 + io * 16 + l], row)
        return 0
    lax.fori_loop(0, E, crow, 0)

    @pl.when(s == 0)
    def _():
        cent_t[pl.ds(0, 16)] = zf
        for uid in range(1, 9):
            cent_t[pl.ds(uid * 16, 16)] = plsc.load_gather(
                centers_v, [io * 16 + uid])
        pltpu.sync_copy(cent_t, centers_out.at[pl.ds(b * CEN, CEN)])
        part_v[pl.ds(0, 16)] = wsum_vec - 1e-8
        part_v[pl.ds(16, 16)] = cnt_vec
        pltpu.sync_copy(part_v.at[pl.ds(0, MISC)],
                        miscred_out.at[pl.ds(b * MISC, MISC)])

    for r in range(16):
        rep[pl.ds(r * 16, 16)] = zf

    for t in range(NSEG):
        k, h = t // 2, t % 2
        z = z0 + k
        if h == 0:
            pltpu.sync_copy(label.at[pl.ds((b * Z + z) * PLANE, PLANE)],
                            labs[0].at[pl.ds(0, PLANE)])
            pltpu.sync_copy(w_out.at[pl.ds((b * Z + z) * PLANE, PLANE)],
                            wbuf)
        for cp in cps:
            cp.wait()
        if t + 1 < NSEG:
            cps = fire(t + 1)
        embx = embs[t % 2]
        hb = h * HPLANE

        def pchunk(i, _, _hb=hb, _embx=embx):
            for u in range(1):
                o = i * 16
                po = _hb + o
                labv = labs[0][pl.ds(po, 16)]
                wv = wbuf[pl.ds(po, 16)]
                base = labv * 16 + io
                sq = []
                for e in range(E):
                    v = _embx[pl.ds(e * HPLANE + o, 16)]
                    cs = plsc.load_gather(crep, [base + e * 256])
                    d = v - cs
                    sq.append(d * d)
                while len(sq) > 1:
                    sq = [a + bq for a, bq in zip(sq[::2], sq[1::2])]
                dist = _nsqrt(sq[0])
                t_ = jnp.maximum(dist - D_V, 0.0)
                ww = jnp.where(labv == 0, 0.0, wv)
                plsc.addupdate_scatter(rep, [base], t_ * t_ * ww)
            return 0
        lax.fori_loop(0, HPLANE // 16, pchunk, 0)

    acc = zf
    for l in range(16):
        acc = acc + plsc.load_gather(rep, [io * 16 + l])
    part_v[pl.ds(0, 16)] = acc
    pltpu.sync_copy(part_v.at[pl.ds(0, 16)], pull_out.at[pl.ds(wid * 16, 16)])


@functools.partial(
    pl.kernel,
    out_type=[jax.ShapeDtypeStruct((16,), jnp.float32)],
    mesh=_mesh(),
    compiler_params=pltpu.CompilerParams(needs_layout_passes=False),
    scratch_types=[
        pltpu.VMEM((NW * 16,), jnp.float32),
        pltpu.VMEM((B * CEN,), jnp.float32),
        pltpu.VMEM((B * MISC,), jnp.float32),
        pltpu.VMEM((16,), jnp.float32),
        pltpu.SemaphoreType.DMA,
    ],
)
def _kern_c(pull_p, centers, miscred, out, pb, cb, mb, ob, sem):
    c = lax.axis_index("c")
    s = lax.axis_index("s")

    @pl.when((c == 0) & (s == 0))
    def _():
        pltpu.sync_copy(pull_p, pb)
        pltpu.sync_copy(centers, cb)
        pltpu.sync_copy(miscred, mb)
        io = lax.iota(jnp.int32, 16)
        uidmask = (io >= 1) & (io <= 8)
        loss_pull = jnp.float32(0.0)
        loss_push = jnp.float32(0.0)
        loss_norm = jnp.float32(0.0)
        for b in range(B):
            def red(i, acc, _b=b):
                return acc + pb[pl.ds((_b * NS + i) * 16, 16)]
            pull_vec = lax.fori_loop(0, NS, red, jnp.zeros((16,), jnp.float32))
            cnt_vec = mb[pl.ds(b * MISC + 16, 16)]
            q = jnp.where(uidmask, pull_vec / cnt_vec, 0.0)
            loss_pull = loss_pull + jnp.sum(q)
            for i in range(1, 9):
                for j in range(i + 1, 9):
                    d = (cb[pl.ds(b * CEN + i * 16, 16)]
                         - cb[pl.ds(b * CEN + j * 16, 16)])
                    dd = jnp.max(_nsqrt(jnp.full((16,), jnp.sum(d * d))))
                    pp = jnp.maximum(2.0 * D_D - dd, 0.0)
                    loss_push = loss_push + pp * pp * (1.0 / 28.0)
            for i in range(1, 9):
                cv = cb[pl.ds(b * CEN + i * 16, 16)]
                loss_norm = loss_norm + jnp.max(
                    _nsqrt(jnp.full((16,), jnp.sum(cv * cv)))) * 0.125
        total = (W_PULL * loss_pull + W_PUSH * loss_push
                 + W_NORM * loss_norm) * (1.0 / B)
        ob[pl.ds(0, 16)] = jnp.full((16,), total)
        pltpu.sync_copy(ob, out)


def kernel(embed, label):
    embed_f = embed.reshape(-1)
    label_f = label.reshape(-1)
    pull_p, centers, miscred, _w = _kern_ab(embed_f, label_f)
    (out,) = _kern_c(pull_p, centers, miscred)
    return out[0]
